# Initial kernel scaffold; baseline (speedup 1.0000x reference)
import jax
import jax.numpy as jnp
from jax.experimental import pallas as pl


def kernel(data, d, edge_index, W0, U0, G0, W1, U1, G1, W2, U2, G2, fW1, fb1, fW2, fb2, fW3, fb3, fW4, fb4):
    raise NotImplementedError("write your pallas kernel here")


# speedup vs baseline: 9.1346x; 9.1346x over previous
import jax
import jax.numpy as jnp
from jax.experimental import pallas as pl

N_NODES = 50
N_EDGES = 800
EMB = 5
F32 = jnp.float32


def _lrelu(x):
    return jnp.where(x >= 0, x, 0.01 * x)


def _sigmoid(x):
    return 1.0 / (1.0 + jnp.exp(-x))


def _dot(a, b):
    return jax.lax.dot_general(a, b, (((1,), (0,)), ((), ())),
                               preferred_element_type=F32)


def _fused(src_ref, dst_ref, data_ref, dflat_ref,
           W0_ref, U0_ref, G0_ref, W1_ref, U1_ref, G1_ref, W2_ref, U2_ref,
           G2_ref, fW1p_ref, fb1_ref, fW2_ref, fb2_ref, fW3_ref, fb3_ref,
           fW4_ref, fb4_ref, out_ref):
    src = src_ref[...]
    dst = dst_ref[...]
    rows = jax.lax.broadcasted_iota(jnp.int32, (N_NODES, N_EDGES), 0)
    m_dst = (rows == dst).astype(F32)
    m_src = (rows == src).astype(F32)
    A = jax.lax.dot_general(m_dst, m_src, (((1,), (1,)), ((), ())),
                            preferred_element_type=F32)

    x0 = data_ref[...]
    ax0 = _dot(A, x0)
    t0 = ax0 * W0_ref[...]
    h = _lrelu(x0 * U0_ref[...] + t0)
    g = _sigmoid(x0 * G0_ref[...] + t0)
    x = jnp.concatenate([h, g * h], axis=1)

    for W_ref, U_ref, G_ref in ((W1_ref, U1_ref, G1_ref),
                                (W2_ref, U2_ref, G2_ref)):
        ax = _dot(A, x)
        t = _dot(ax, W_ref[...])
        h = _lrelu(_dot(x, U_ref[...]) + t)
        g = _sigmoid(_dot(x, G_ref[...]) + t)
        x = jnp.concatenate([h, g * h], axis=1)

    xt = jnp.transpose(x)
    vecx = jnp.concatenate([xt[j:j + 1, :] for j in range(2 * EMB)], axis=1)
    full = jnp.concatenate([vecx, dflat_ref[...]], axis=1)

    h1 = _lrelu(_dot(full, fW1p_ref[...]) + fb1_ref[...])
    h2 = _lrelu(_dot(h1, fW2_ref[...]) + fb2_ref[...])
    h3 = _lrelu(_dot(h2, fW3_ref[...]) + fb3_ref[...])
    y = _sigmoid(_dot(h3, fW4_ref[...]) + fb4_ref[...])
    out_ref[...] = 0.5 * y[:, 0:1] + 0.5 * y[:, 1:2]


def kernel(data, d, edge_index, W0, U0, G0, W1, U1, G1, W2, U2, G2,
           fW1, fb1, fW2, fb2, fW3, fb3, fW4, fb4):
    src = edge_index[0].reshape(1, N_EDGES)
    dst = edge_index[1].reshape(1, N_EDGES)
    dflat = d.reshape(1, -1)
    fW1x = fW1[:N_NODES * 2 * EMB].reshape(N_NODES, 2 * EMB, -1)
    fW1p = jnp.concatenate(
        [fW1x.transpose(1, 0, 2).reshape(N_NODES * 2 * EMB, -1),
         fW1[N_NODES * 2 * EMB:]], axis=0)
    out = pl.pallas_call(
        _fused,
        out_shape=jax.ShapeDtypeStruct((1, 1), F32),
    )(src, dst, data, dflat, W0, U0, G0, W1, U1, G1, W2, U2, G2,
      fW1p, fb1.reshape(1, -1), fW2, fb2.reshape(1, -1),
      fW3, fb3.reshape(1, -1), fW4, fb4.reshape(1, -1))
    return out[0, 0]
